# baseline (speedup 1.0000x reference)
import functools

import jax
import jax.numpy as jnp
from jax import lax
from jax.experimental import pallas as pl
from jax.experimental.pallas import tpu as pltpu

_DEPTH = 4
_WIDTH = 16
_CPAD = 8
_H = 32
_W = 32
_HW = _H * _W
_NOUT = 10
_NOUT_PAD = 16
_BT = 16
_S = _BT * _W


def _fused_kernel(x_ref, w0_ref, b0_ref, wres_ref, bres_ref, wlin_ref,
                  blin_ref, msk_ref, o_ref, *, bt):
    lanes = bt * _HW
    f32 = jnp.float32
    bf16 = jnp.bfloat16

    m_colL = msk_ref[0:1]
    m_colR = msk_ref[1:2]

    def conv3x3(act, w_bf, b, cin):
        a_bf = act.astype(bf16)
        a_i = pltpu.bitcast(a_bf, jnp.int32)
        ap = pltpu.bitcast(pltpu.roll(a_i, 1, axis=1), bf16) * m_colL
        am = pltpu.bitcast(pltpu.roll(a_i, lanes - 1, axis=1), bf16) * m_colR
        cs = jnp.concatenate([ap, a_bf, am], axis=0)
        k3 = 3 * cin
        w_dn = w_bf[:, 0:k3]
        w_md = w_bf[:, k3:2 * k3]
        w_up = w_bf[:, 2 * k3:3 * k3]
        p0 = jnp.dot(w_md, cs, preferred_element_type=f32)
        pu = jnp.dot(w_up, cs[:, _S:], preferred_element_type=f32)
        pd = jnp.dot(w_dn, cs[:, :lanes - _S], preferred_element_type=f32)
        z = jnp.zeros((_WIDTH, _S), f32)
        y = p0 + jnp.concatenate([pu, z], axis=1) \
               + jnp.concatenate([z, pd], axis=1)
        return y + b

    act = conv3x3(x_ref[0], w0_ref[...], b0_ref[...], _CPAD)
    for i in range(_DEPTH):
        y = conv3x3(act, wres_ref[i], bres_ref[i], _WIDTH)
        act = jnp.maximum(y, 0.0) + act

    act_bf = act.astype(bf16)
    wl = wlin_ref[...]
    rows = []
    for b in range(bt):
        ab = jnp.concatenate(
            [act_bf[:, r * _S + b * _W: r * _S + b * _W + _W]
             for r in range(_H)], axis=1)
        full = jnp.dot(ab, wl, preferred_element_type=f32)
        r = full[0:1, 0:_NOUT_PAD]
        for c in range(1, _WIDTH):
            r = r + full[c:c + 1, c * _NOUT_PAD:(c + 1) * _NOUT_PAD]
        rows.append(r)
    o_ref[0] = jnp.concatenate(rows, axis=0) + blin_ref[...]


def kernel(x, w0, b0, wres, bres, wlin, blin):
    N, Cin, H, W = x.shape
    bt = _BT
    n_pad = pl.cdiv(N, bt) * bt
    G = n_pad // bt
    lanes = bt * _HW

    x = x.reshape(N, Cin, _HW)
    if _CPAD > Cin:
        x = jnp.pad(x, ((0, 0), (0, _CPAD - Cin), (0, 0)))
    if n_pad > N:
        x = jnp.pad(x, ((0, n_pad - N), (0, 0), (0, 0)))
    x = x.reshape(G, bt, _CPAD, _H, _W).transpose(0, 2, 3, 1, 4)
    x = x.reshape(G, _CPAD, lanes)

    lane = jnp.arange(lanes, dtype=jnp.int32)
    col = lane % _W
    msk = jnp.stack([col >= 1, col <= _W - 2]).astype(jnp.bfloat16)

    w0_bf = w0.astype(jnp.bfloat16)
    wres_bf = wres.astype(jnp.bfloat16)
    wlin_bf = wlin.astype(jnp.bfloat16)

    kern = functools.partial(_fused_kernel, bt=bt)
    const = pl.Buffered(1)
    out = pl.pallas_call(
        kern,
        out_shape=jax.ShapeDtypeStruct((G, bt, _NOUT_PAD), jnp.float32),
        grid_spec=pltpu.PrefetchScalarGridSpec(
            num_scalar_prefetch=0,
            grid=(G,),
            in_specs=[
                pl.BlockSpec((1, _CPAD, lanes), lambda g: (g, 0, 0)),
                pl.BlockSpec((_WIDTH, 9 * _CPAD), lambda g: (0, 0),
                             pipeline_mode=const),
                pl.BlockSpec((_WIDTH, 1), lambda g: (0, 0), pipeline_mode=const),
                pl.BlockSpec((_DEPTH, _WIDTH, 9 * _WIDTH), lambda g: (0, 0, 0),
                             pipeline_mode=const),
                pl.BlockSpec((_DEPTH, _WIDTH, 1), lambda g: (0, 0, 0),
                             pipeline_mode=const),
                pl.BlockSpec((_HW, _WIDTH * _NOUT_PAD), lambda g: (0, 0),
                             pipeline_mode=const),
                pl.BlockSpec((1, _NOUT_PAD), lambda g: (0, 0),
                             pipeline_mode=const),
                pl.BlockSpec((2, lanes), lambda g: (0, 0), pipeline_mode=const),
            ],
            out_specs=pl.BlockSpec((1, bt, _NOUT_PAD), lambda g: (g, 0, 0)),
        ),
        compiler_params=pltpu.CompilerParams(
            dimension_semantics=("parallel",),
            vmem_limit_bytes=48 * 1024 * 1024,
        ),
    )(x, w0_bf, b0, wres_bf, bres, wlin_bf, blin, msk)
    return out.reshape(n_pad, _NOUT_PAD)[:N, :_NOUT]


# speedup vs baseline: 1.5786x; 1.5786x over previous
import functools

import jax
import jax.numpy as jnp
from jax import lax
from jax.experimental import pallas as pl
from jax.experimental.pallas import tpu as pltpu

_DEPTH = 4
_WIDTH = 16
_CIN = 3
_H = 32
_W = 32
_HW = _H * _W
_NOUT = 10
_NOUT_PAD = 16
_BT = 16
_S = _BT * _W


def _conv_dots(cs, w_bf, b, k3, lanes):
    f32 = jnp.float32
    w_dn = w_bf[:, 0:k3]
    w_md = w_bf[:, k3:2 * k3]
    w_up = w_bf[:, 2 * k3:3 * k3]
    p0 = jnp.dot(w_md, cs, preferred_element_type=f32)
    pu = jnp.dot(w_up, cs[:, _S:], preferred_element_type=f32)
    pd = jnp.dot(w_dn, cs[:, :lanes - _S], preferred_element_type=f32)
    z = jnp.zeros((_WIDTH, _S), f32)
    return p0 + jnp.concatenate([pu, z], axis=1) \
              + jnp.concatenate([z, pd], axis=1) + b


def _fused_kernel(x_ref, w0_ref, b0_ref, wres_ref, bres_ref, wlin_ref,
                  blin_ref, msk_ref, o_ref, *, bt):
    lanes = bt * _HW
    f32 = jnp.float32
    bf16 = jnp.bfloat16

    m_colL = msk_ref[0:1]
    m_colR = msk_ref[1:2]

    xall = x_ref[0]
    xi = jnp.concatenate(
        [xall[b, :, r * _W:(r + 1) * _W]
         for r in range(_H) for b in range(bt)], axis=1)

    ap = pltpu.roll(xi, 1, axis=1).astype(bf16) * m_colL
    am = pltpu.roll(xi, lanes - 1, axis=1).astype(bf16) * m_colR
    cs0 = jnp.concatenate([ap, xi.astype(bf16), am], axis=0)
    act = _conv_dots(cs0, w0_ref[...], b0_ref[...], 3 * _CIN, lanes)

    for i in range(_DEPTH):
        a_bf = act.astype(bf16)
        a_i = pltpu.bitcast(a_bf, jnp.int32)
        rp = pltpu.bitcast(pltpu.roll(a_i, 1, axis=1), bf16) * m_colL
        rm = pltpu.bitcast(pltpu.roll(a_i, lanes - 1, axis=1), bf16) * m_colR
        cs = jnp.concatenate([rp, a_bf, rm], axis=0)
        y = _conv_dots(cs, wres_ref[i], bres_ref[i], 3 * _WIDTH, lanes)
        act = jnp.maximum(y, 0.0) + act

    act_bf = act.astype(bf16)
    wl = wlin_ref[...]
    rows = []
    for b in range(bt):
        ab = jnp.concatenate(
            [act_bf[:, r * _S + b * _W: r * _S + b * _W + _W]
             for r in range(_H)], axis=1)
        full = jnp.dot(ab, wl, preferred_element_type=f32)
        r = full[0:1, 0:_NOUT_PAD]
        for c in range(1, _WIDTH):
            r = r + full[c:c + 1, c * _NOUT_PAD:(c + 1) * _NOUT_PAD]
        rows.append(r)
    o_ref[0] = jnp.concatenate(rows, axis=0) + blin_ref[...]


def kernel(x, w0, b0, wres, bres, wlin, blin):
    N, Cin, H, W = x.shape
    bt = _BT
    n_pad = pl.cdiv(N, bt) * bt
    G = n_pad // bt
    lanes = bt * _HW

    x = x.reshape(N, Cin, _HW)
    if n_pad > N:
        x = jnp.pad(x, ((0, n_pad - N), (0, 0), (0, 0)))
    x = x.reshape(G, bt, Cin, _HW)

    lane = jnp.arange(lanes, dtype=jnp.int32)
    col = lane % _W
    msk = jnp.stack([col >= 1, col <= _W - 2]).astype(jnp.bfloat16)

    w0_bf = w0.reshape(_WIDTH, 9, 8)[:, :, :Cin].reshape(
        _WIDTH, 9 * Cin).astype(jnp.bfloat16)
    wres_bf = wres.astype(jnp.bfloat16)
    wlin_bf = wlin.astype(jnp.bfloat16)

    kern = functools.partial(_fused_kernel, bt=bt)
    const = pl.Buffered(1)
    out = pl.pallas_call(
        kern,
        out_shape=jax.ShapeDtypeStruct((G, bt, _NOUT_PAD), jnp.float32),
        grid_spec=pltpu.PrefetchScalarGridSpec(
            num_scalar_prefetch=0,
            grid=(G,),
            in_specs=[
                pl.BlockSpec((1, bt, Cin, _HW), lambda g: (g, 0, 0, 0)),
                pl.BlockSpec((_WIDTH, 9 * Cin), lambda g: (0, 0),
                             pipeline_mode=const),
                pl.BlockSpec((_WIDTH, 1), lambda g: (0, 0), pipeline_mode=const),
                pl.BlockSpec((_DEPTH, _WIDTH, 9 * _WIDTH), lambda g: (0, 0, 0),
                             pipeline_mode=const),
                pl.BlockSpec((_DEPTH, _WIDTH, 1), lambda g: (0, 0, 0),
                             pipeline_mode=const),
                pl.BlockSpec((_HW, _WIDTH * _NOUT_PAD), lambda g: (0, 0),
                             pipeline_mode=const),
                pl.BlockSpec((1, _NOUT_PAD), lambda g: (0, 0),
                             pipeline_mode=const),
                pl.BlockSpec((2, lanes), lambda g: (0, 0), pipeline_mode=const),
            ],
            out_specs=pl.BlockSpec((1, bt, _NOUT_PAD), lambda g: (g, 0, 0)),
        ),
        compiler_params=pltpu.CompilerParams(
            dimension_semantics=("parallel",),
            vmem_limit_bytes=48 * 1024 * 1024,
        ),
    )(x, w0_bf, b0, wres_bf, bres, wlin_bf, blin, msk)
    return out.reshape(n_pad, _NOUT_PAD)[:N, :_NOUT]


# speedup vs baseline: 1.9372x; 1.2272x over previous
import functools

import jax
import jax.numpy as jnp
from jax import lax
from jax.experimental import pallas as pl
from jax.experimental.pallas import tpu as pltpu

_DEPTH = 4
_WIDTH = 16
_CIN = 3
_H = 32
_W = 32
_HW = _H * _W
_NOUT = 10
_NOUT_PAD = 16
_BT = 16
_S = _BT * _W


def _conv_dots(cs, w3, b, lanes):
    f32 = jnp.float32
    P = jnp.dot(w3, cs, preferred_element_type=f32)
    z = jnp.zeros((_WIDTH, _S), f32)
    return P[_WIDTH:2 * _WIDTH] \
        + jnp.concatenate([P[2 * _WIDTH:, _S:], z], axis=1) \
        + jnp.concatenate([z, P[:_WIDTH, :lanes - _S]], axis=1) + b


def _fused_kernel(x_ref, w0_ref, b0_ref, wres_ref, bres_ref, wlin_ref,
                  blin_ref, msk_ref, o_ref, *, bt):
    lanes = bt * _HW
    f32 = jnp.float32
    bf16 = jnp.bfloat16

    m_colL = msk_ref[0:1]
    m_colR = msk_ref[1:2]

    xall = x_ref[0]
    xi = jnp.concatenate(
        [xall[b, :, r * _W:(r + 1) * _W]
         for r in range(_H) for b in range(bt)], axis=1)

    ap = pltpu.roll(xi, 1, axis=1).astype(bf16) * m_colL
    am = pltpu.roll(xi, lanes - 1, axis=1).astype(bf16) * m_colR
    cs0 = jnp.concatenate([ap, xi.astype(bf16), am], axis=0)
    act = _conv_dots(cs0, w0_ref[...], b0_ref[...], lanes)

    for i in range(_DEPTH):
        a_bf = act.astype(bf16)
        a_i = pltpu.bitcast(a_bf, jnp.int32)
        rp = pltpu.bitcast(pltpu.roll(a_i, 1, axis=1), bf16) * m_colL
        rm = pltpu.bitcast(pltpu.roll(a_i, lanes - 1, axis=1), bf16) * m_colR
        cs = jnp.concatenate([rp, a_bf, rm], axis=0)
        y = _conv_dots(cs, wres_ref[i], bres_ref[i], lanes)
        act = jnp.maximum(y, 0.0) + act

    act_bf = act.astype(bf16)
    ab_all = jnp.concatenate(
        [act_bf[:, r * _S + b * _W: r * _S + b * _W + _W]
         for b in range(bt) for r in range(_H)], axis=1)
    AB = jnp.concatenate(
        [ab_all[:, b * _HW:(b + 1) * _HW] for b in range(bt)],
        axis=0)
    full = jnp.dot(AB, wlin_ref[...], preferred_element_type=f32)
    rows = []
    for b in range(bt):
        r = full[b * _WIDTH: b * _WIDTH + 1, 0:_NOUT_PAD]
        for c in range(1, _WIDTH):
            r = r + full[b * _WIDTH + c: b * _WIDTH + c + 1,
                         c * _NOUT_PAD:(c + 1) * _NOUT_PAD]
        rows.append(r)
    o_ref[0] = jnp.concatenate(rows, axis=0) + blin_ref[...]


def kernel(x, w0, b0, wres, bres, wlin, blin):
    N, Cin, H, W = x.shape
    bt = _BT
    n_pad = pl.cdiv(N, bt) * bt
    G = n_pad // bt
    lanes = bt * _HW

    x = x.reshape(N, Cin, _HW)
    if n_pad > N:
        x = jnp.pad(x, ((0, n_pad - N), (0, 0), (0, 0)))
    x = x.reshape(G, bt, Cin, _HW)

    lane = jnp.arange(lanes, dtype=jnp.int32)
    col = lane % _W
    msk = jnp.stack([col >= 1, col <= _W - 2]).astype(jnp.bfloat16)

    w0_r = w0.reshape(_WIDTH, 9, 8)[:, :, :Cin].reshape(_WIDTH, 9 * Cin)
    k30 = 3 * Cin
    w0_bf = jnp.concatenate(
        [w0_r[:, 0:k30], w0_r[:, k30:2 * k30], w0_r[:, 2 * k30:3 * k30]],
        axis=0).astype(jnp.bfloat16)
    k3 = 3 * _WIDTH
    wres_bf = jnp.concatenate(
        [wres[:, :, 0:k3], wres[:, :, k3:2 * k3], wres[:, :, 2 * k3:3 * k3]],
        axis=1).astype(jnp.bfloat16)
    wlin_bf = wlin.astype(jnp.bfloat16)

    kern = functools.partial(_fused_kernel, bt=bt)
    const = pl.Buffered(1)
    out = pl.pallas_call(
        kern,
        out_shape=jax.ShapeDtypeStruct((G, bt, _NOUT_PAD), jnp.float32),
        grid_spec=pltpu.PrefetchScalarGridSpec(
            num_scalar_prefetch=0,
            grid=(G,),
            in_specs=[
                pl.BlockSpec((1, bt, Cin, _HW), lambda g: (g, 0, 0, 0)),
                pl.BlockSpec((3 * _WIDTH, 3 * Cin), lambda g: (0, 0),
                             pipeline_mode=const),
                pl.BlockSpec((_WIDTH, 1), lambda g: (0, 0), pipeline_mode=const),
                pl.BlockSpec((_DEPTH, 3 * _WIDTH, 3 * _WIDTH),
                             lambda g: (0, 0, 0), pipeline_mode=const),
                pl.BlockSpec((_DEPTH, _WIDTH, 1), lambda g: (0, 0, 0),
                             pipeline_mode=const),
                pl.BlockSpec((_HW, _WIDTH * _NOUT_PAD), lambda g: (0, 0),
                             pipeline_mode=const),
                pl.BlockSpec((1, _NOUT_PAD), lambda g: (0, 0),
                             pipeline_mode=const),
                pl.BlockSpec((2, lanes), lambda g: (0, 0), pipeline_mode=const),
            ],
            out_specs=pl.BlockSpec((1, bt, _NOUT_PAD), lambda g: (g, 0, 0)),
        ),
        compiler_params=pltpu.CompilerParams(
            dimension_semantics=("parallel",),
            vmem_limit_bytes=48 * 1024 * 1024,
        ),
    )(x, w0_bf, b0, wres_bf, bres, wlin_bf, blin, msk)
    return out.reshape(n_pad, _NOUT_PAD)[:N, :_NOUT]


# speedup vs baseline: 2.1908x; 1.1309x over previous
import functools

import jax
import jax.numpy as jnp
from jax import lax
from jax.experimental import pallas as pl
from jax.experimental.pallas import tpu as pltpu

_DEPTH = 4
_WIDTH = 16
_CIN = 3
_H = 32
_W = 32
_HW = _H * _W
_NOUT = 10
_NOUT_PAD = 16
_BT = 32
_S = _BT * _W


def _conv_dots(cs, w3, b, lanes):
    f32 = jnp.float32
    P = jnp.dot(w3, cs, preferred_element_type=f32)
    z = jnp.zeros((_WIDTH, _S), f32)
    return P[_WIDTH:2 * _WIDTH] \
        + jnp.concatenate([P[2 * _WIDTH:, _S:], z], axis=1) \
        + jnp.concatenate([z, P[:_WIDTH, :lanes - _S]], axis=1) + b


def _fused_kernel(x_ref, w0_ref, b0_ref, wres_ref, bres_ref, wlin_ref,
                  blin_ref, msk_ref, o_ref, *, bt):
    lanes = bt * _HW
    f32 = jnp.float32
    bf16 = jnp.bfloat16

    m_colL = msk_ref[0:1]
    m_colR = msk_ref[1:2]

    xall = x_ref[0]
    xi = jnp.concatenate(
        [xall[b, :, r * _W:(r + 1) * _W]
         for r in range(_H) for b in range(bt)], axis=1)

    ap = pltpu.roll(xi, 1, axis=1).astype(bf16) * m_colL
    am = pltpu.roll(xi, lanes - 1, axis=1).astype(bf16) * m_colR
    cs0 = jnp.concatenate([ap, xi.astype(bf16), am], axis=0)
    act = _conv_dots(cs0, w0_ref[...], b0_ref[...], lanes)

    for i in range(_DEPTH):
        a_bf = act.astype(bf16)
        a_i = pltpu.bitcast(a_bf, jnp.int32)
        rp = pltpu.bitcast(pltpu.roll(a_i, 1, axis=1), bf16) * m_colL
        rm = pltpu.bitcast(pltpu.roll(a_i, lanes - 1, axis=1), bf16) * m_colR
        cs = jnp.concatenate([rp, a_bf, rm], axis=0)
        y = _conv_dots(cs, wres_ref[i], bres_ref[i], lanes)
        act = jnp.maximum(y, 0.0) + act

    act_bf = act.astype(bf16)
    ab_all = jnp.concatenate(
        [act_bf[:, r * _S + b * _W: r * _S + b * _W + _W]
         for b in range(bt) for r in range(_H)], axis=1)
    AB = jnp.concatenate(
        [ab_all[:, b * _HW:(b + 1) * _HW] for b in range(bt)],
        axis=0)
    full = jnp.dot(AB, wlin_ref[...], preferred_element_type=f32)
    rows = []
    for b in range(bt):
        r = full[b * _WIDTH: b * _WIDTH + 1, 0:_NOUT_PAD]
        for c in range(1, _WIDTH):
            r = r + full[b * _WIDTH + c: b * _WIDTH + c + 1,
                         c * _NOUT_PAD:(c + 1) * _NOUT_PAD]
        rows.append(r)
    o_ref[0] = jnp.concatenate(rows, axis=0) + blin_ref[...]


def kernel(x, w0, b0, wres, bres, wlin, blin):
    N, Cin, H, W = x.shape
    bt = _BT
    n_pad = pl.cdiv(N, bt) * bt
    G = n_pad // bt
    lanes = bt * _HW

    x = x.reshape(N, Cin, _HW)
    if n_pad > N:
        x = jnp.pad(x, ((0, n_pad - N), (0, 0), (0, 0)))
    x = x.reshape(G, bt, Cin, _HW)

    lane = jnp.arange(lanes, dtype=jnp.int32)
    col = lane % _W
    msk = jnp.stack([col >= 1, col <= _W - 2]).astype(jnp.bfloat16)

    w0_r = w0.reshape(_WIDTH, 9, 8)[:, :, :Cin].reshape(_WIDTH, 9 * Cin)
    k30 = 3 * Cin
    w0_bf = jnp.concatenate(
        [w0_r[:, 0:k30], w0_r[:, k30:2 * k30], w0_r[:, 2 * k30:3 * k30]],
        axis=0).astype(jnp.bfloat16)
    k3 = 3 * _WIDTH
    wres_bf = jnp.concatenate(
        [wres[:, :, 0:k3], wres[:, :, k3:2 * k3], wres[:, :, 2 * k3:3 * k3]],
        axis=1).astype(jnp.bfloat16)
    wlin_bf = wlin.astype(jnp.bfloat16)

    kern = functools.partial(_fused_kernel, bt=bt)
    const = pl.Buffered(1)
    out = pl.pallas_call(
        kern,
        out_shape=jax.ShapeDtypeStruct((G, bt, _NOUT_PAD), jnp.float32),
        grid_spec=pltpu.PrefetchScalarGridSpec(
            num_scalar_prefetch=0,
            grid=(G,),
            in_specs=[
                pl.BlockSpec((1, bt, Cin, _HW), lambda g: (g, 0, 0, 0)),
                pl.BlockSpec((3 * _WIDTH, 3 * Cin), lambda g: (0, 0),
                             pipeline_mode=const),
                pl.BlockSpec((_WIDTH, 1), lambda g: (0, 0), pipeline_mode=const),
                pl.BlockSpec((_DEPTH, 3 * _WIDTH, 3 * _WIDTH),
                             lambda g: (0, 0, 0), pipeline_mode=const),
                pl.BlockSpec((_DEPTH, _WIDTH, 1), lambda g: (0, 0, 0),
                             pipeline_mode=const),
                pl.BlockSpec((_HW, _WIDTH * _NOUT_PAD), lambda g: (0, 0),
                             pipeline_mode=const),
                pl.BlockSpec((1, _NOUT_PAD), lambda g: (0, 0),
                             pipeline_mode=const),
                pl.BlockSpec((2, lanes), lambda g: (0, 0), pipeline_mode=const),
            ],
            out_specs=pl.BlockSpec((1, bt, _NOUT_PAD), lambda g: (g, 0, 0)),
        ),
        compiler_params=pltpu.CompilerParams(
            dimension_semantics=("parallel",),
            vmem_limit_bytes=48 * 1024 * 1024,
        ),
    )(x, w0_bf, b0, wres_bf, bres, wlin_bf, blin, msk)
    return out.reshape(n_pad, _NOUT_PAD)[:N, :_NOUT]


# speedup vs baseline: 2.3757x; 1.0844x over previous
import functools

import jax
import jax.numpy as jnp
from jax import lax
from jax.experimental import pallas as pl
from jax.experimental.pallas import tpu as pltpu

_DEPTH = 4
_WIDTH = 16
_CIN = 3
_H = 32
_W = 32
_HW = _H * _W
_NOUT = 10
_NOUT_PAD = 16
_BT = 32
_S = _BT * _W


def _conv_dots(cs, w3, b, lanes):
    f32 = jnp.float32
    P = jnp.dot(w3, cs, preferred_element_type=f32)
    z = jnp.zeros((_WIDTH, _S), f32)
    return P[_WIDTH:2 * _WIDTH] \
        + jnp.concatenate([P[2 * _WIDTH:, _S:], z], axis=1) \
        + jnp.concatenate([z, P[:_WIDTH, :lanes - _S]], axis=1) + b


def _fused_kernel(x_ref, w0_ref, b0_ref, wres_ref, bres_ref, wlin_ref,
                  blin_ref, msk_ref, o_ref, *, bt):
    lanes = bt * _HW
    f32 = jnp.float32
    bf16 = jnp.bfloat16

    m_colL = msk_ref[0:1]
    m_colR = msk_ref[1:2]

    xall = x_ref[0]
    xi = jnp.concatenate(
        [xall[b * _CIN:(b + 1) * _CIN, r * _W:(r + 1) * _W]
         for r in range(_H) for b in range(bt)], axis=1)

    ap = pltpu.roll(xi, 1, axis=1).astype(bf16) * m_colL
    am = pltpu.roll(xi, lanes - 1, axis=1).astype(bf16) * m_colR
    cs0 = jnp.concatenate([ap, xi.astype(bf16), am], axis=0)
    act = _conv_dots(cs0, w0_ref[...], b0_ref[...], lanes)

    for i in range(_DEPTH):
        a_bf = act.astype(bf16)
        a_i = pltpu.bitcast(a_bf, jnp.int32)
        rp = pltpu.bitcast(pltpu.roll(a_i, 1, axis=1), bf16) * m_colL
        rm = pltpu.bitcast(pltpu.roll(a_i, lanes - 1, axis=1), bf16) * m_colR
        cs = jnp.concatenate([rp, a_bf, rm], axis=0)
        y = _conv_dots(cs, wres_ref[i], bres_ref[i], lanes)
        act = jnp.maximum(y, 0.0) + act

    act_bf = act.astype(bf16)
    ab_all = jnp.concatenate(
        [act_bf[:, r * _S + b * _W: r * _S + b * _W + _W]
         for b in range(bt) for r in range(_H)], axis=1)
    AB = jnp.concatenate(
        [ab_all[:, b * _HW:(b + 1) * _HW] for b in range(bt)],
        axis=0)
    full = jnp.dot(AB, wlin_ref[...], preferred_element_type=f32)
    rows = []
    for b in range(bt):
        r = full[b * _WIDTH: b * _WIDTH + 1, 0:_NOUT_PAD]
        for c in range(1, _WIDTH):
            r = r + full[b * _WIDTH + c: b * _WIDTH + c + 1,
                         c * _NOUT_PAD:(c + 1) * _NOUT_PAD]
        rows.append(r)
    o_ref[0] = jnp.concatenate(rows, axis=0) + blin_ref[...]


def kernel(x, w0, b0, wres, bres, wlin, blin):
    N, Cin, H, W = x.shape
    bt = _BT
    n_pad = pl.cdiv(N, bt) * bt
    G = n_pad // bt
    lanes = bt * _HW

    x = x.reshape(N, Cin, _HW)
    if n_pad > N:
        x = jnp.pad(x, ((0, n_pad - N), (0, 0), (0, 0)))
    x = x.reshape(G, bt * Cin, _HW)

    lane = jnp.arange(lanes, dtype=jnp.int32)
    col = lane % _W
    msk = jnp.stack([col >= 1, col <= _W - 2]).astype(jnp.bfloat16)

    w0_r = w0.reshape(_WIDTH, 9, 8)[:, :, :Cin].reshape(_WIDTH, 9 * Cin)
    k30 = 3 * Cin
    w0_bf = jnp.concatenate(
        [w0_r[:, 0:k30], w0_r[:, k30:2 * k30], w0_r[:, 2 * k30:3 * k30]],
        axis=0).astype(jnp.bfloat16)
    k3 = 3 * _WIDTH
    wres_bf = jnp.concatenate(
        [wres[:, :, 0:k3], wres[:, :, k3:2 * k3], wres[:, :, 2 * k3:3 * k3]],
        axis=1).astype(jnp.bfloat16)
    wlin_bf = wlin.astype(jnp.bfloat16)

    kern = functools.partial(_fused_kernel, bt=bt)
    const = pl.Buffered(1)
    out = pl.pallas_call(
        kern,
        out_shape=jax.ShapeDtypeStruct((G, bt, _NOUT_PAD), jnp.float32),
        grid_spec=pltpu.PrefetchScalarGridSpec(
            num_scalar_prefetch=0,
            grid=(G,),
            in_specs=[
                pl.BlockSpec((1, bt * Cin, _HW), lambda g: (g, 0, 0)),
                pl.BlockSpec((3 * _WIDTH, 3 * Cin), lambda g: (0, 0),
                             pipeline_mode=const),
                pl.BlockSpec((_WIDTH, 1), lambda g: (0, 0), pipeline_mode=const),
                pl.BlockSpec((_DEPTH, 3 * _WIDTH, 3 * _WIDTH),
                             lambda g: (0, 0, 0), pipeline_mode=const),
                pl.BlockSpec((_DEPTH, _WIDTH, 1), lambda g: (0, 0, 0),
                             pipeline_mode=const),
                pl.BlockSpec((_HW, _WIDTH * _NOUT_PAD), lambda g: (0, 0),
                             pipeline_mode=const),
                pl.BlockSpec((1, _NOUT_PAD), lambda g: (0, 0),
                             pipeline_mode=const),
                pl.BlockSpec((2, lanes), lambda g: (0, 0), pipeline_mode=const),
            ],
            out_specs=pl.BlockSpec((1, bt, _NOUT_PAD), lambda g: (g, 0, 0)),
        ),
        compiler_params=pltpu.CompilerParams(
            dimension_semantics=("parallel",),
            vmem_limit_bytes=48 * 1024 * 1024,
        ),
    )(x, w0_bf, b0, wres_bf, bres, wlin_bf, blin, msk)
    return out.reshape(n_pad, _NOUT_PAD)[:N, :_NOUT]


# speedup vs baseline: 2.3776x; 1.0008x over previous
import functools

import jax
import jax.numpy as jnp
from jax import lax
from jax.experimental import pallas as pl
from jax.experimental.pallas import tpu as pltpu

_DEPTH = 4
_WIDTH = 16
_CIN = 3
_H = 32
_W = 32
_HW = _H * _W
_NOUT = 10
_NOUT_PAD = 16
_BT = 32
_S = _BT * _W


def _conv_dots(cs, w3, b, lanes):
    f32 = jnp.float32
    P = jnp.dot(w3, cs, preferred_element_type=f32)
    z = jnp.zeros((_WIDTH, _S), f32)
    return P[_WIDTH:2 * _WIDTH] \
        + jnp.concatenate([P[2 * _WIDTH:, _S:], z], axis=1) \
        + jnp.concatenate([z, P[:_WIDTH, :lanes - _S]], axis=1) + b


def _fused_kernel(x_ref, w0_ref, b0_ref, wres_ref, bres_ref, wlin_ref,
                  blin_ref, msk_ref, o_ref, *, bt):
    lanes = bt * _HW
    f32 = jnp.float32
    bf16 = jnp.bfloat16

    m_colL = msk_ref[0:1]
    m_colR = msk_ref[1:2]

    xall = x_ref[0].astype(f32)
    xi = jnp.concatenate(
        [xall[b * _CIN:(b + 1) * _CIN, r * _W:(r + 1) * _W]
         for r in range(_H) for b in range(bt)], axis=1)

    ap = pltpu.roll(xi, 1, axis=1).astype(bf16) * m_colL
    am = pltpu.roll(xi, lanes - 1, axis=1).astype(bf16) * m_colR
    cs0 = jnp.concatenate([ap, xi.astype(bf16), am], axis=0)
    act = _conv_dots(cs0, w0_ref[...], b0_ref[...], lanes)

    for i in range(_DEPTH):
        a_bf = act.astype(bf16)
        a_i = pltpu.bitcast(a_bf, jnp.int32)
        rp = pltpu.bitcast(pltpu.roll(a_i, 1, axis=1), bf16) * m_colL
        rm = pltpu.bitcast(pltpu.roll(a_i, lanes - 1, axis=1), bf16) * m_colR
        cs = jnp.concatenate([rp, a_bf, rm], axis=0)
        y = _conv_dots(cs, wres_ref[i], bres_ref[i], lanes)
        act = jnp.maximum(y, 0.0) + act

    act_bf = act.astype(bf16)
    ab_all = jnp.concatenate(
        [act_bf[:, r * _S + b * _W: r * _S + b * _W + _W]
         for b in range(bt) for r in range(_H)], axis=1)
    AB = jnp.concatenate(
        [ab_all[:, b * _HW:(b + 1) * _HW] for b in range(bt)],
        axis=0)
    full = jnp.dot(AB, wlin_ref[...], preferred_element_type=f32)
    rows = []
    for b in range(bt):
        r = full[b * _WIDTH: b * _WIDTH + 1, 0:_NOUT_PAD]
        for c in range(1, _WIDTH):
            r = r + full[b * _WIDTH + c: b * _WIDTH + c + 1,
                         c * _NOUT_PAD:(c + 1) * _NOUT_PAD]
        rows.append(r)
    o_ref[0] = jnp.concatenate(rows, axis=0) + blin_ref[...]


def kernel(x, w0, b0, wres, bres, wlin, blin):
    N, Cin, H, W = x.shape
    bt = _BT
    n_pad = pl.cdiv(N, bt) * bt
    G = n_pad // bt
    lanes = bt * _HW

    x = x.reshape(N, Cin, _HW)
    if n_pad > N:
        x = jnp.pad(x, ((0, n_pad - N), (0, 0), (0, 0)))
    x = x.reshape(G, bt * Cin, _HW).astype(jnp.bfloat16)

    lane = jnp.arange(lanes, dtype=jnp.int32)
    col = lane % _W
    msk = jnp.stack([col >= 1, col <= _W - 2]).astype(jnp.bfloat16)

    w0_r = w0.reshape(_WIDTH, 9, 8)[:, :, :Cin].reshape(_WIDTH, 9 * Cin)
    k30 = 3 * Cin
    w0_bf = jnp.concatenate(
        [w0_r[:, 0:k30], w0_r[:, k30:2 * k30], w0_r[:, 2 * k30:3 * k30]],
        axis=0).astype(jnp.bfloat16)
    k3 = 3 * _WIDTH
    wres_bf = jnp.concatenate(
        [wres[:, :, 0:k3], wres[:, :, k3:2 * k3], wres[:, :, 2 * k3:3 * k3]],
        axis=1).astype(jnp.bfloat16)
    wlin_bf = wlin.astype(jnp.bfloat16)

    kern = functools.partial(_fused_kernel, bt=bt)
    const = pl.Buffered(1)
    out = pl.pallas_call(
        kern,
        out_shape=jax.ShapeDtypeStruct((G, bt, _NOUT_PAD), jnp.float32),
        grid_spec=pltpu.PrefetchScalarGridSpec(
            num_scalar_prefetch=0,
            grid=(G,),
            in_specs=[
                pl.BlockSpec((1, bt * Cin, _HW), lambda g: (g, 0, 0)),
                pl.BlockSpec((3 * _WIDTH, 3 * Cin), lambda g: (0, 0),
                             pipeline_mode=const),
                pl.BlockSpec((_WIDTH, 1), lambda g: (0, 0), pipeline_mode=const),
                pl.BlockSpec((_DEPTH, 3 * _WIDTH, 3 * _WIDTH),
                             lambda g: (0, 0, 0), pipeline_mode=const),
                pl.BlockSpec((_DEPTH, _WIDTH, 1), lambda g: (0, 0, 0),
                             pipeline_mode=const),
                pl.BlockSpec((_HW, _WIDTH * _NOUT_PAD), lambda g: (0, 0),
                             pipeline_mode=const),
                pl.BlockSpec((1, _NOUT_PAD), lambda g: (0, 0),
                             pipeline_mode=const),
                pl.BlockSpec((2, lanes), lambda g: (0, 0), pipeline_mode=const),
            ],
            out_specs=pl.BlockSpec((1, bt, _NOUT_PAD), lambda g: (g, 0, 0)),
        ),
        compiler_params=pltpu.CompilerParams(
            dimension_semantics=("parallel",),
            vmem_limit_bytes=48 * 1024 * 1024,
        ),
    )(x, w0_bf, b0, wres_bf, bres, wlin_bf, blin, msk)
    return out.reshape(n_pad, _NOUT_PAD)[:N, :_NOUT]


# speedup vs baseline: 2.6051x; 1.0957x over previous
import functools

import jax
import jax.numpy as jnp
from jax import lax
from jax.experimental import pallas as pl
from jax.experimental.pallas import tpu as pltpu

_DEPTH = 4
_WIDTH = 16
_CIN = 3
_H = 32
_W = 32
_HW = _H * _W
_NOUT = 10
_NOUT_PAD = 16
_BT = 64
_S = _BT * _W


def _conv_dots(cs, w3, b, lanes):
    f32 = jnp.float32
    P = jnp.dot(w3, cs, preferred_element_type=f32)
    z = jnp.zeros((_WIDTH, _S), f32)
    return P[_WIDTH:2 * _WIDTH] \
        + jnp.concatenate([P[2 * _WIDTH:, _S:], z], axis=1) \
        + jnp.concatenate([z, P[:_WIDTH, :lanes - _S]], axis=1) + b


def _fused_kernel(x_ref, w0_ref, b0_ref, wres_ref, bres_ref, wlin_ref,
                  blin_ref, msk_ref, o_ref, *, bt):
    lanes = bt * _HW
    f32 = jnp.float32
    bf16 = jnp.bfloat16

    m_colL = msk_ref[0:1]
    m_colR = msk_ref[1:2]

    xall = x_ref[0].astype(f32)
    xi = jnp.concatenate(
        [xall[b * _CIN:(b + 1) * _CIN, r * _W:(r + 1) * _W]
         for r in range(_H) for b in range(bt)], axis=1)

    ap = pltpu.roll(xi, 1, axis=1).astype(bf16) * m_colL
    am = pltpu.roll(xi, lanes - 1, axis=1).astype(bf16) * m_colR
    cs0 = jnp.concatenate([ap, xi.astype(bf16), am], axis=0)
    act = _conv_dots(cs0, w0_ref[...], b0_ref[...], lanes)

    for i in range(_DEPTH):
        a_bf = act.astype(bf16)
        a_i = pltpu.bitcast(a_bf, jnp.int32)
        rp = pltpu.bitcast(pltpu.roll(a_i, 1, axis=1), bf16) * m_colL
        rm = pltpu.bitcast(pltpu.roll(a_i, lanes - 1, axis=1), bf16) * m_colR
        cs = jnp.concatenate([rp, a_bf, rm], axis=0)
        y = _conv_dots(cs, wres_ref[i], bres_ref[i], lanes)
        act = jnp.maximum(y, 0.0) + act

    act_bf = act.astype(bf16)
    ab_all = jnp.concatenate(
        [act_bf[:, r * _S + b * _W: r * _S + b * _W + _W]
         for b in range(bt) for r in range(_H)], axis=1)
    AB = jnp.concatenate(
        [ab_all[:, b * _HW:(b + 1) * _HW] for b in range(bt)],
        axis=0)
    full = jnp.dot(AB, wlin_ref[...], preferred_element_type=f32)
    rows = []
    for b in range(bt):
        r = full[b * _WIDTH: b * _WIDTH + 1, 0:_NOUT_PAD]
        for c in range(1, _WIDTH):
            r = r + full[b * _WIDTH + c: b * _WIDTH + c + 1,
                         c * _NOUT_PAD:(c + 1) * _NOUT_PAD]
        rows.append(r)
    o_ref[0] = jnp.concatenate(rows, axis=0) + blin_ref[...]


def kernel(x, w0, b0, wres, bres, wlin, blin):
    N, Cin, H, W = x.shape
    bt = _BT
    n_pad = pl.cdiv(N, bt) * bt
    G = n_pad // bt
    lanes = bt * _HW

    x = x.reshape(N, Cin, _HW)
    if n_pad > N:
        x = jnp.pad(x, ((0, n_pad - N), (0, 0), (0, 0)))
    x = x.reshape(G, bt * Cin, _HW).astype(jnp.bfloat16)

    lane = jnp.arange(lanes, dtype=jnp.int32)
    col = lane % _W
    msk = jnp.stack([col >= 1, col <= _W - 2]).astype(jnp.bfloat16)

    w0_r = w0.reshape(_WIDTH, 9, 8)[:, :, :Cin].reshape(_WIDTH, 9 * Cin)
    k30 = 3 * Cin
    w0_bf = jnp.concatenate(
        [w0_r[:, 0:k30], w0_r[:, k30:2 * k30], w0_r[:, 2 * k30:3 * k30]],
        axis=0).astype(jnp.bfloat16)
    k3 = 3 * _WIDTH
    wres_bf = jnp.concatenate(
        [wres[:, :, 0:k3], wres[:, :, k3:2 * k3], wres[:, :, 2 * k3:3 * k3]],
        axis=1).astype(jnp.bfloat16)
    wlin_bf = wlin.astype(jnp.bfloat16)

    kern = functools.partial(_fused_kernel, bt=bt)
    const = pl.Buffered(1)
    out = pl.pallas_call(
        kern,
        out_shape=jax.ShapeDtypeStruct((G, bt, _NOUT_PAD), jnp.float32),
        grid_spec=pltpu.PrefetchScalarGridSpec(
            num_scalar_prefetch=0,
            grid=(G,),
            in_specs=[
                pl.BlockSpec((1, bt * Cin, _HW), lambda g: (g, 0, 0)),
                pl.BlockSpec((3 * _WIDTH, 3 * Cin), lambda g: (0, 0),
                             pipeline_mode=const),
                pl.BlockSpec((_WIDTH, 1), lambda g: (0, 0), pipeline_mode=const),
                pl.BlockSpec((_DEPTH, 3 * _WIDTH, 3 * _WIDTH),
                             lambda g: (0, 0, 0), pipeline_mode=const),
                pl.BlockSpec((_DEPTH, _WIDTH, 1), lambda g: (0, 0, 0),
                             pipeline_mode=const),
                pl.BlockSpec((_HW, _WIDTH * _NOUT_PAD), lambda g: (0, 0),
                             pipeline_mode=const),
                pl.BlockSpec((1, _NOUT_PAD), lambda g: (0, 0),
                             pipeline_mode=const),
                pl.BlockSpec((2, lanes), lambda g: (0, 0), pipeline_mode=const),
            ],
            out_specs=pl.BlockSpec((1, bt, _NOUT_PAD), lambda g: (g, 0, 0)),
        ),
        compiler_params=pltpu.CompilerParams(
            dimension_semantics=("parallel",),
            vmem_limit_bytes=56 * 1024 * 1024,
        ),
    )(x, w0_bf, b0, wres_bf, bres, wlin_bf, blin, msk)
    return out.reshape(n_pad, _NOUT_PAD)[:N, :_NOUT]
